# baseline (speedup 1.0000x reference)
import functools

import jax
import jax.numpy as jnp
from jax import lax
from jax.experimental import pallas as pl
from jax.experimental.pallas import tpu as pltpu
from jax.experimental.pallas import tpu_sc as plsc

H = 512
HH = H * H
RB = 256
NB = H // RB

NC = 2
NS = 16
NW = NC * NS
CHUNK = HH // NW
NVEC = CHUNK // 16
INV255 = 1.0 / 255.0


def _tc_body(inp_ref, tgt_ref, ms_ref, o2_ref, o3_ref, loss_ref):
    rb = pl.program_id(0)
    c = pl.program_id(1)

    ms = ms_ref[...] * INV255
    td = jnp.clip((tgt_ref[0] + 1.0) * 0.5, 0.0, 1.0)
    idt = jnp.clip(inp_ref[0], 0.0, 1.0)

    o2 = idt * ms
    o3 = td * ms
    o2_ref[0] = o2
    o3_ref[0] = o3

    part = jnp.sum(jnp.abs(o2 - o3))

    @pl.when((rb == 0) & (c == 0))
    def _():
        loss_ref[0] = 0.0

    loss_ref[0] += part


def _sc_body(tgt_hbm, ref_hbm, ms_hbm, mt_hbm, o0_hbm, o1_hbm,
             tgt_v, ref_v, ms_v, mt_v,
             sem_m, sem_c0, sem_c1, sem_c2, sem_out):
    wid = lax.axis_index("s") * NC + lax.axis_index("c")
    off = wid * CHUNK
    sem_c = (sem_c0, sem_c1, sem_c2)

    cp_ms = pltpu.async_copy(ms_hbm.at[pl.ds(off, CHUNK)], ms_v, sem_m)
    cp_mt = pltpu.async_copy(mt_hbm.at[pl.ds(off, CHUNK)], mt_v, sem_m)
    cins = []
    for c in range(3):
        base = c * HH + off
        vsl = pl.ds(c * CHUNK, CHUNK)
        cins.append((
            pltpu.async_copy(tgt_hbm.at[pl.ds(base, CHUNK)], tgt_v.at[vsl], sem_c[c]),
            pltpu.async_copy(ref_hbm.at[pl.ds(base, CHUNK)], ref_v.at[vsl], sem_c[c]),
        ))
    cp_ms.wait()
    cp_mt.wait()

    couts = []
    for c in range(3):
        cins[c][0].wait()
        cins[c][1].wait()

        @plsc.parallel_loop(0, CHUNK, step=16, unroll=8)
        def _(i, c=c):
            sl = pl.ds(c * CHUNK + i, 16)
            msl = pl.ds(i, 16)
            td = jnp.clip(tgt_v[sl] * 0.5 + 0.5, 0.0, 1.0)
            tgt_v[sl] = td * (mt_v[msl] * INV255)
            rf = jnp.clip(ref_v[sl] * 0.5 + 0.5, 0.0, 1.0)
            ref_v[sl] = rf * (ms_v[msl] * INV255)

        base = c * HH + off
        vsl = pl.ds(c * CHUNK, CHUNK)
        couts.append(pltpu.async_copy(tgt_v.at[vsl], o0_hbm.at[pl.ds(base, CHUNK)], sem_out))
        couts.append(pltpu.async_copy(ref_v.at[vsl], o1_hbm.at[pl.ds(base, CHUNK)], sem_out))

    for cp in couts:
        cp.wait()


_sc_call = functools.partial(
    pl.kernel,
    mesh=plsc.VectorSubcoreMesh(core_axis_name="c", subcore_axis_name="s"),
    out_type=[
        jax.ShapeDtypeStruct((3 * HH,), jnp.float32),
        jax.ShapeDtypeStruct((3 * HH,), jnp.float32),
    ],
    scratch_types=[
        pltpu.VMEM((3 * CHUNK,), jnp.float32),
        pltpu.VMEM((3 * CHUNK,), jnp.float32),
        pltpu.VMEM((CHUNK,), jnp.float32),
        pltpu.VMEM((CHUNK,), jnp.float32),
        pltpu.SemaphoreType.DMA,
        pltpu.SemaphoreType.DMA,
        pltpu.SemaphoreType.DMA,
        pltpu.SemaphoreType.DMA,
        pltpu.SemaphoreType.DMA,
    ],
)(_sc_body)


def kernel(input_data, target_data, target_data_eye, mask_src, mask_tar, index, ref):
    del target_data_eye, index
    inp = input_data.reshape(3, H, H)
    tgt = target_data.reshape(3, H, H)
    ms = mask_src.reshape(H, H)

    img_spec = pl.BlockSpec((1, RB, H), lambda rb, c: (c, rb, 0))
    mask_spec = pl.BlockSpec((RB, H), lambda rb, c: (rb, 0))

    o2, o3, loss = pl.pallas_call(
        _tc_body,
        grid=(NB, 3),
        in_specs=[img_spec, img_spec, mask_spec],
        out_specs=[img_spec, img_spec, pl.BlockSpec(memory_space=pltpu.SMEM)],
        out_shape=[
            jax.ShapeDtypeStruct((3, H, H), jnp.float32),
            jax.ShapeDtypeStruct((3, H, H), jnp.float32),
            jax.ShapeDtypeStruct((1,), jnp.float32),
        ],
    )(inp, tgt, ms)

    o0_flat, o1_flat = _sc_call(
        target_data.reshape(3 * HH),
        ref.reshape(3 * HH),
        mask_src.reshape(HH),
        mask_tar.reshape(HH),
    )

    n = jnp.float32(3 * HH)
    return (
        o0_flat.reshape(1, 3, H, H),
        o1_flat.reshape(1, 3, H, H),
        o2.reshape(1, 3, H, H),
        o3.reshape(1, 3, H, H),
        loss[0] / n,
    )


# speedup vs baseline: 3.0501x; 3.0501x over previous
import jax
import jax.numpy as jnp
from jax.experimental import pallas as pl
from jax.experimental.pallas import tpu as pltpu

H = 512
RB = 256
NB = H // RB


def _body(inp_ref, tgt_ref, ref_ref, ms_ref, mt_ref,
          o0_ref, o1_ref, o2_ref, o3_ref, loss_ref):
    rb = pl.program_id(0)
    c = pl.program_id(1)

    ms = ms_ref[...] * (1.0 / 255.0)
    mt = mt_ref[...] * (1.0 / 255.0)

    td = jnp.clip((tgt_ref[0] + 1.0) * 0.5, 0.0, 1.0) * 255.0
    rf = jnp.clip((ref_ref[0] + 1.0) * 0.5, 0.0, 1.0) * 255.0
    idt = jnp.clip(inp_ref[0], 0.0, 1.0) * 255.0

    inv255 = 1.0 / 255.0
    o0 = (td * mt) * inv255
    o1 = (rf * ms) * inv255
    o2 = (idt * ms) * inv255
    o3 = (td * ms) * inv255

    o0_ref[0] = o0
    o1_ref[0] = o1
    o2_ref[0] = o2
    o3_ref[0] = o3

    loss_ref[rb * 3 + c] = jnp.sum(jnp.abs(o2 - o3))


def kernel(input_data, target_data, target_data_eye, mask_src, mask_tar, index, ref):
    del target_data_eye, index
    inp = input_data.reshape(3, H, H)
    tgt = target_data.reshape(3, H, H)
    rf = ref.reshape(3, H, H)
    ms = mask_src.reshape(H, H)
    mt = mask_tar.reshape(H, H)

    img_spec = pl.BlockSpec((1, RB, H), lambda rb, c: (c, rb, 0))
    mask_spec = pl.BlockSpec((RB, H), lambda rb, c: (rb, 0))

    out_shapes = (
        [jax.ShapeDtypeStruct((3, H, H), jnp.float32)] * 4
        + [jax.ShapeDtypeStruct((NB * 3,), jnp.float32)]
    )
    out_specs = (
        [img_spec] * 4
        + [pl.BlockSpec(memory_space=pltpu.SMEM)]
    )

    o0, o1, o2, o3, loss = pl.pallas_call(
        _body,
        grid=(NB, 3),
        in_specs=[img_spec, img_spec, img_spec, mask_spec, mask_spec],
        out_specs=out_specs,
        out_shape=out_shapes,
    )(inp, tgt, rf, ms, mt)

    n = jnp.float32(3 * H * H)
    return (
        o0.reshape(1, 3, H, H),
        o1.reshape(1, 3, H, H),
        o2.reshape(1, 3, H, H),
        o3.reshape(1, 3, H, H),
        jnp.sum(loss) / n,
    )
